# Initial kernel scaffold; baseline (speedup 1.0000x reference)
import functools

import jax
import jax.numpy as jnp
from jax import lax
from jax.experimental import pallas as pl
from jax.experimental.pallas import tpu as pltpu
from jax.experimental.pallas import tpu_sc as plsc

_NUM_FIELDS = 26
_VOCAB = 100000
_EMBED_DIM = 32
_BATCH = 16384
_TOTAL = _BATCH * _NUM_FIELDS

_LANES = 16
_NC = 2
_NS = 16
_NW = _NC * _NS

_CHUNK = 128
_NBUF = 8
_N_CHUNKS = _TOTAL // _CHUNK
_CHUNKS_W = _N_CHUNKS // _NW


def _build():
    mesh = plsc.VectorSubcoreMesh(core_axis_name="c", subcore_axis_name="s")

    @functools.partial(
        pl.kernel,
        mesh=mesh,
        out_type=jax.ShapeDtypeStruct((_TOTAL, _EMBED_DIM), jnp.float32),
        scratch_types=[
            pltpu.VMEM((_CHUNKS_W, _CHUNK), jnp.int32),
            pltpu.VMEM((_NBUF, _CHUNK, _EMBED_DIM), jnp.float32),
            pltpu.SemaphoreType.DMA,
            pltpu.SemaphoreType.DMA,
        ],
    )
    def gather_kernel(w_hbm, x_hbm, out_hbm, idx_v, rows_v, sem_g, sem_w):
        wid = lax.axis_index("s") * _NC + lax.axis_index("c")
        c0 = wid * _CHUNKS_W

        pltpu.sync_copy(x_hbm.at[pl.ds(c0, _CHUNKS_W)], idx_v)

        lane = lax.iota(jnp.int32, _LANES)

        def fixup(j):
            for s in range(_CHUNK // _LANES):
                base = (c0 + j) * _CHUNK + s * _LANES
                fld = lax.rem(base + lane, _NUM_FIELDS)
                sl = pl.ds(s * _LANES, _LANES)
                idx_v[j, sl] = idx_v[j, sl] + fld * _VOCAB

        def fire_gather(j, b):
            pltpu.async_copy(w_hbm.at[idx_v.at[j]], rows_v.at[b], sem_g)

        def wait_gather(j, b):
            pltpu.make_async_copy(w_hbm.at[idx_v.at[j]], rows_v.at[b], sem_g).wait()

        for b in range(_NBUF):
            fixup(b)
            fire_gather(b, b)

        def body(i, carry):
            j0 = i * _NBUF
            for b in range(_NBUF):
                j = j0 + b
                wait_gather(j, b)
                dst = out_hbm.at[pl.ds((c0 + j) * _CHUNK, _CHUNK)]
                pltpu.async_copy(rows_v.at[b], dst, sem_w)
                pltpu.make_async_copy(rows_v.at[b], dst, sem_w).wait()

                @pl.when(j + _NBUF < _CHUNKS_W)
                def _():
                    fixup(j + _NBUF)
                    fire_gather(j + _NBUF, b)

            return carry

        lax.fori_loop(0, _CHUNKS_W // _NBUF, body, 0)

    return gather_kernel


_GATHER = _build()


def kernel(x, W):
    w_flat = W.reshape(_NUM_FIELDS * _VOCAB, _EMBED_DIM)
    x_flat = x.reshape(_N_CHUNKS, _CHUNK)
    out = _GATHER(w_flat, x_flat)
    return out.reshape(_BATCH, _NUM_FIELDS, _EMBED_DIM)


# speedup vs baseline: 1.1542x; 1.1542x over previous
import functools

import jax
import jax.numpy as jnp
from jax import lax
from jax.experimental import pallas as pl
from jax.experimental.pallas import tpu as pltpu
from jax.experimental.pallas import tpu_sc as plsc

_NUM_FIELDS = 26
_VOCAB = 100000
_EMBED_DIM = 32
_BATCH = 16384
_TOTAL = _BATCH * _NUM_FIELDS

_LANES = 16
_NC = 2
_NS = 16
_NW = _NC * _NS

_CHUNK = 128
_NBUF = 8
_N_CHUNKS = _TOTAL // _CHUNK
_CHUNKS_W = _N_CHUNKS // _NW


def _build():
    mesh = plsc.VectorSubcoreMesh(core_axis_name="c", subcore_axis_name="s")

    @functools.partial(
        pl.kernel,
        mesh=mesh,
        out_type=jax.ShapeDtypeStruct((_TOTAL, _EMBED_DIM), jnp.float32),
        scratch_types=[
            pltpu.VMEM((_CHUNKS_W, _CHUNK), jnp.int32),
            pltpu.VMEM((_NBUF, _CHUNK, _EMBED_DIM), jnp.float32),
            pltpu.SemaphoreType.DMA,
            pltpu.SemaphoreType.DMA,
        ],
        compiler_params=pltpu.CompilerParams(use_tc_tiling_on_sc=False),
    )
    def gather_kernel(w_hbm, x_hbm, out_hbm, idx_v, rows_v, sem_g, sem_w):
        wid = lax.axis_index("s") * _NC + lax.axis_index("c")
        c0 = wid * _CHUNKS_W

        pltpu.sync_copy(x_hbm.at[pl.ds(c0, _CHUNKS_W)], idx_v)

        lane = lax.iota(jnp.int32, _LANES)

        def fixup(j):
            for s in range(_CHUNK // _LANES):
                base = (c0 + j) * _CHUNK + s * _LANES
                fld = lax.rem(base + lane, _NUM_FIELDS)
                sl = pl.ds(s * _LANES, _LANES)
                idx_v[j, sl] = idx_v[j, sl] + fld * _VOCAB

        def fire_gather(j, b):
            pltpu.async_copy(w_hbm.at[idx_v.at[j]], rows_v.at[b], sem_g)

        def wait_gather(j, b):
            pltpu.make_async_copy(w_hbm.at[idx_v.at[j]], rows_v.at[b], sem_g).wait()

        for b in range(_NBUF):
            fixup(b)
            fire_gather(b, b)

        def body(i, carry):
            j0 = i * _NBUF
            for b in range(_NBUF):
                j = j0 + b
                wait_gather(j, b)
                dst = out_hbm.at[pl.ds((c0 + j) * _CHUNK, _CHUNK)]
                pltpu.async_copy(rows_v.at[b], dst, sem_w)
                pltpu.make_async_copy(rows_v.at[b], dst, sem_w).wait()

                @pl.when(j + _NBUF < _CHUNKS_W)
                def _():
                    fixup(j + _NBUF)
                    fire_gather(j + _NBUF, b)

            return carry

        lax.fori_loop(0, _CHUNKS_W // _NBUF, body, 0)

    return gather_kernel


_GATHER = _build()


def kernel(x, W):
    w_flat = W.reshape(_NUM_FIELDS * _VOCAB, _EMBED_DIM)
    x_flat = x.reshape(_N_CHUNKS, _CHUNK)
    out = _GATHER(w_flat, x_flat)
    return out.reshape(_BATCH, _NUM_FIELDS, _EMBED_DIM)


# speedup vs baseline: 3.8513x; 3.3369x over previous
import functools

import jax
import jax.numpy as jnp
from jax import lax
from jax.experimental import pallas as pl
from jax.experimental.pallas import tpu as pltpu
from jax.experimental.pallas import tpu_sc as plsc

_NUM_FIELDS = 26
_VOCAB = 100000
_EMBED_DIM = 32
_BATCH = 16384

_LANES = 16
_NC = 2
_NS = 16
_NW = _NC * _NS

_BCHUNK = 4096
_NCHUNK = _BATCH // _BCHUNK


def _build():
    mesh = plsc.VectorSubcoreMesh(core_axis_name="c", subcore_axis_name="s")

    @functools.partial(
        pl.kernel,
        mesh=mesh,
        out_type=jax.ShapeDtypeStruct((_NUM_FIELDS, _EMBED_DIM, _BATCH), jnp.float32),
        scratch_types=[
            pltpu.VMEM((_VOCAB,), jnp.float32),
            pltpu.VMEM((_BATCH,), jnp.int32),
            pltpu.VMEM((2, _BCHUNK), jnp.float32),
        ],
        compiler_params=pltpu.CompilerParams(
            use_tc_tiling_on_sc=True, needs_layout_passes=False
        ),
    )
    def lookup_kernel(w_hbm, x_hbm, out_hbm, row_v, xbuf, obuf):
        d = lax.axis_index("s") * _NC + lax.axis_index("c")

        def per_field(f, carry):
            pltpu.sync_copy(w_hbm.at[f, d], row_v)
            pltpu.sync_copy(x_hbm.at[f], xbuf)
            for c in range(_NCHUNK):
                b = c % 2

                def gather16(i, carry2):
                    sl = pl.ds(i * _LANES, _LANES)
                    idx = xbuf[pl.ds(c * _BCHUNK + i * _LANES, _LANES)]
                    obuf[b, sl] = plsc.load_gather(row_v, [idx])
                    return carry2

                lax.fori_loop(0, _BCHUNK // _LANES, gather16, 0)
                pltpu.sync_copy(
                    obuf.at[b], out_hbm.at[f, d, pl.ds(c * _BCHUNK, _BCHUNK)]
                )
            return carry

        lax.fori_loop(0, _NUM_FIELDS, per_field, 0)

    return lookup_kernel


_LOOKUP = _build()


def kernel(x, W):
    w_t = jnp.transpose(W, (0, 2, 1))
    x_t = jnp.transpose(x, (1, 0))
    out_t = _LOOKUP(w_t, x_t)
    return jnp.transpose(out_t, (2, 0, 1))
